# baseline (speedup 1.0000x reference)
import functools

import jax
import jax.numpy as jnp
from jax import lax
from jax.experimental import pallas as pl
from jax.experimental.pallas import tpu as pltpu
from jax.experimental.pallas import tpu_sc as plsc

B = 16384
LT = 20
LD = 200
V = 100000
D = 64
C = 100
H = 128

NC = 2
NS = 16
NW = NC * NS
RPW = B // NW
DCH = 8
NCH = LD // DCH


def _scores_body(et_ref, w2_ref, out_ref):
    s = lax.dot_general(
        w2_ref[...], et_ref[...],
        dimension_numbers=(((1,), (0,)), ((), ())),
        preferred_element_type=jnp.float32)
    st = lax.bitcast_convert_type(
        s[0:1, :].astype(jnp.bfloat16), jnp.uint16).astype(jnp.uint32)
    sd = lax.bitcast_convert_type(
        s[1:2, :].astype(jnp.bfloat16), jnp.uint16).astype(jnp.uint32)
    packed = (st << 16) | sd
    out_ref[...] = lax.bitcast_convert_type(packed, jnp.int32)


def _compute_scores(ET, w2):
    VB = 12800
    return pl.pallas_call(
        _scores_body,
        grid=(pl.cdiv(V, VB),),
        in_specs=[
            pl.BlockSpec((D, VB), lambda i: (0, i)),
            pl.BlockSpec((2, D), lambda i: (0, 0)),
        ],
        out_specs=pl.BlockSpec((1, VB), lambda i: (0, i)),
        out_shape=jax.ShapeDtypeStruct((1, V), jnp.int32),
    )(ET, w2)


def _sc_pool(s2, titleT, descT):
    mesh = plsc.VectorSubcoreMesh(core_axis_name="c", subcore_axis_name="s")

    @functools.partial(
        pl.kernel,
        mesh=mesh,
        out_type=jax.ShapeDtypeStruct((B,), jnp.float32),
        compiler_params=pltpu.CompilerParams(needs_layout_passes=False),
        scratch_types=[
            pltpu.VMEM((V,), jnp.int32),
            pltpu.VMEM((LT, RPW), jnp.int32),
            pltpu.VMEM((DCH, RPW), jnp.int32),
            pltpu.VMEM((DCH, RPW), jnp.int32),
            pltpu.VMEM((RPW,), jnp.float32),
            pltpu.SemaphoreType.DMA,
            pltpu.SemaphoreType.DMA,
            pltpu.SemaphoreType.DMA,
        ],
    )
    def k(s2_hbm, titleT_hbm, descT_hbm, out_hbm, table_v, tidx_v, buf_a,
          buf_b, acc_v, semT, semA, semB):
        w = lax.axis_index("c") * NS + lax.axis_index("s")
        r0 = w * RPW
        himask = jnp.full((16,), -65536, jnp.int32)

        tcp = pltpu.make_async_copy(s2_hbm.at[0], table_v, semT)
        tcp.start()
        ti = pltpu.make_async_copy(
            titleT_hbm.at[:, pl.ds(r0, RPW)], tidx_v, semT)
        ti.start()
        pltpu.make_async_copy(
            descT_hbm.at[pl.ds(0, DCH), pl.ds(r0, RPW)], buf_a,
            semA).start()
        pltpu.make_async_copy(
            descT_hbm.at[pl.ds(DCH, DCH), pl.ds(r0, RPW)], buf_b,
            semB).start()
        tcp.wait()
        ti.wait()

        def title_group(g, carry):
            acc0 = jnp.zeros((16,), jnp.float32)
            acc1 = jnp.zeros((16,), jnp.float32)
            for t in range(LT):
                iv = tidx_v[t, pl.ds(g * 16, 16)]
                word = plsc.load_gather(table_v, [iv])
                vals = plsc.bitcast(word & himask, jnp.float32)
                if t % 2 == 0:
                    acc0 = acc0 + vals
                else:
                    acc1 = acc1 + vals
            acc_v[pl.ds(g * 16, 16)] = acc0 + acc1
            return carry

        lax.fori_loop(0, RPW // 16, title_group, 0)

        def desc_chunk(buf):
            def group(g, c2):
                acc0 = jnp.zeros((16,), jnp.float32)
                acc1 = jnp.zeros((16,), jnp.float32)
                for t in range(DCH):
                    iv = buf[t, pl.ds(g * 16, 16)]
                    word = plsc.load_gather(table_v, [iv])
                    vals = plsc.bitcast(word << 16, jnp.float32)
                    if t % 2 == 0:
                        acc0 = acc0 + vals
                    else:
                        acc1 = acc1 + vals
                acc_v[pl.ds(g * 16, 16)] = (acc_v[pl.ds(g * 16, 16)]
                                            + (acc0 + acc1))
                return c2

            lax.fori_loop(0, RPW // 16, group, 0)

        def desc_chunk2(g, c2):
            accs = [jnp.zeros((16,), jnp.float32) for _ in range(4)]
            for t in range(DCH):
                iv = buf_a[t, pl.ds(g * 16, 16)]
                word = plsc.load_gather(table_v, [iv])
                accs[t % 4] = accs[t % 4] + plsc.bitcast(word << 16,
                                                         jnp.float32)
            for t in range(DCH):
                iv = buf_b[t, pl.ds(g * 16, 16)]
                word = plsc.load_gather(table_v, [iv])
                accs[t % 4] = accs[t % 4] + plsc.bitcast(word << 16,
                                                         jnp.float32)
            tot = (accs[0] + accs[1]) + (accs[2] + accs[3])
            acc_v[pl.ds(g * 16, 16)] = acc_v[pl.ds(g * 16, 16)] + tot
            return c2

        def pair(p, carry):
            c = 2 * p
            pltpu.make_async_copy(
                descT_hbm.at[pl.ds(0, DCH), pl.ds(r0, RPW)], buf_a,
                semA).wait()
            pltpu.make_async_copy(
                descT_hbm.at[pl.ds(0, DCH), pl.ds(r0, RPW)], buf_b,
                semB).wait()
            lax.fori_loop(0, RPW // 16, desc_chunk2, 0)
            pltpu.make_async_copy(
                descT_hbm.at[pl.ds((c + 2) * DCH, DCH), pl.ds(r0, RPW)],
                buf_a, semA).start()

            @pl.when(p < NCH // 2 - 1)
            def _():
                pltpu.make_async_copy(
                    descT_hbm.at[pl.ds((c + 3) * DCH, DCH),
                                 pl.ds(r0, RPW)], buf_b, semB).start()

            return carry

        lax.fori_loop(0, NCH // 2, pair, 0)

        pltpu.make_async_copy(
            descT_hbm.at[pl.ds(0, DCH), pl.ds(r0, RPW)], buf_a,
            semA).wait()
        desc_chunk(buf_a)

        pltpu.sync_copy(acc_v, out_hbm.at[pl.ds(r0, RPW)])

    return k(s2, titleT, descT)


def _cat_body(catT_ref, wcatT_ref, bcat_ref, wh_ref, bout_ref, out_ref):
    h = lax.dot_general(
        wcatT_ref[...], catT_ref[...],
        dimension_numbers=(((0,), (0,)), ((), ())),
        preferred_element_type=jnp.float32)
    h = jnp.maximum(h + bcat_ref[...], 0.0)
    o = lax.dot_general(
        wh_ref[...], h,
        dimension_numbers=(((1,), (0,)), ((), ())),
        preferred_element_type=jnp.float32)
    out_ref[...] = o + bout_ref[0, 0]


def _cat_part(CatT, WcatT, b_cat2, wh2, b_out2):
    BB = 2048
    return pl.pallas_call(
        _cat_body,
        grid=(B // BB,),
        in_specs=[
            pl.BlockSpec((C, BB), lambda i: (0, i)),
            pl.BlockSpec((C, H), lambda i: (0, 0)),
            pl.BlockSpec((H, 1), lambda i: (0, 0)),
            pl.BlockSpec((1, H), lambda i: (0, 0)),
            pl.BlockSpec((1, 1), lambda i: (0, 0)),
        ],
        out_specs=pl.BlockSpec((1, BB), lambda i: (0, i)),
        out_shape=jax.ShapeDtypeStruct((1, B), jnp.float32),
    )(CatT, WcatT, b_cat2, wh2, b_out2)


def _add_body(a_ref, b_ref, out_ref):
    out_ref[...] = a_ref[...] + b_ref[...]


def _add_final(cat2, emb2):
    BB = 8192
    return pl.pallas_call(
        _add_body,
        grid=(B // BB,),
        in_specs=[
            pl.BlockSpec((1, BB), lambda i: (0, i)),
            pl.BlockSpec((1, BB), lambda i: (0, i)),
        ],
        out_specs=pl.BlockSpec((1, BB), lambda i: (0, i)),
        out_shape=jax.ShapeDtypeStruct((1, B), jnp.float32),
    )(cat2, emb2)


def kernel(Title, FullDescription, Categorical, embedding_matrix, W_cat,
           b_cat, W_out, b_out):
    w2 = jnp.stack([W_out[0, :D] * (1.0 / LT),
                    W_out[0, D:2 * D] * (1.0 / LD)], axis=0)
    s2 = _compute_scores(embedding_matrix.T, w2)

    emb_part = _sc_pool(s2, Title.T.astype(jnp.int32),
                        FullDescription.T.astype(jnp.int32))

    cat2 = _cat_part(Categorical.T, W_cat.T, b_cat.reshape(H, 1),
                     W_out[0:1, 2 * D:], b_out.reshape(1, 1))
    out2 = _add_final(cat2, emb_part.reshape(1, B))
    return out2.reshape(B)


# speedup vs baseline: 1.0712x; 1.0712x over previous
import functools

import jax
import jax.numpy as jnp
from jax import lax
from jax.experimental import pallas as pl
from jax.experimental.pallas import tpu as pltpu
from jax.experimental.pallas import tpu_sc as plsc

B = 16384
LT = 20
LD = 200
V = 100000
D = 64
C = 100
H = 128

NC = 2
NS = 16
NW = NC * NS
RPW = B // NW
DCH = 8
NCH = LD // DCH


def _scores_body(et_ref, w2_ref, out_ref):
    s = lax.dot_general(
        w2_ref[...], et_ref[...],
        dimension_numbers=(((1,), (0,)), ((), ())),
        preferred_element_type=jnp.float32)
    st = lax.bitcast_convert_type(
        s[0:1, :].astype(jnp.bfloat16), jnp.uint16).astype(jnp.uint32)
    sd = lax.bitcast_convert_type(
        s[1:2, :].astype(jnp.bfloat16), jnp.uint16).astype(jnp.uint32)
    packed = (st << 16) | sd
    out_ref[...] = lax.bitcast_convert_type(packed, jnp.int32)


def _compute_scores(ET, w2):
    VB = 12800
    return pl.pallas_call(
        _scores_body,
        grid=(pl.cdiv(V, VB),),
        in_specs=[
            pl.BlockSpec((D, VB), lambda i: (0, i)),
            pl.BlockSpec((2, D), lambda i: (0, 0)),
        ],
        out_specs=pl.BlockSpec((1, VB), lambda i: (0, i)),
        out_shape=jax.ShapeDtypeStruct((1, V), jnp.int32),
    )(ET, w2)


def _sc_pool(s2, titleT, descT):
    mesh = plsc.VectorSubcoreMesh(core_axis_name="c", subcore_axis_name="s")

    @functools.partial(
        pl.kernel,
        mesh=mesh,
        out_type=jax.ShapeDtypeStruct((B,), jnp.float32),
        compiler_params=pltpu.CompilerParams(needs_layout_passes=False),
        scratch_types=[
            pltpu.VMEM((V,), jnp.int32),
            pltpu.VMEM((LT, RPW), jnp.int32),
            pltpu.VMEM((DCH, RPW), jnp.int32),
            pltpu.VMEM((DCH, RPW), jnp.int32),
            pltpu.VMEM((RPW,), jnp.float32),
            pltpu.SemaphoreType.DMA,
            pltpu.SemaphoreType.DMA,
            pltpu.SemaphoreType.DMA,
        ],
    )
    def k(s2_hbm, titleT_hbm, descT_hbm, out_hbm, table_v, tidx_v, buf_a,
          buf_b, acc_v, semT, semA, semB):
        w = lax.axis_index("c") * NS + lax.axis_index("s")
        r0 = w * RPW
        himask = jnp.full((16,), -65536, jnp.int32)

        tcp = pltpu.make_async_copy(s2_hbm.at[0], table_v, semT)
        tcp.start()
        ti = pltpu.make_async_copy(
            titleT_hbm.at[:, pl.ds(r0, RPW)], tidx_v, semT)
        ti.start()
        pltpu.make_async_copy(
            descT_hbm.at[pl.ds(0, DCH), pl.ds(r0, RPW)], buf_a,
            semA).start()
        pltpu.make_async_copy(
            descT_hbm.at[pl.ds(DCH, DCH), pl.ds(r0, RPW)], buf_b,
            semB).start()
        tcp.wait()
        ti.wait()

        def title_group(g, carry):
            acc0 = jnp.zeros((16,), jnp.float32)
            acc1 = jnp.zeros((16,), jnp.float32)
            for t in range(LT):
                iv = tidx_v[t, pl.ds(g * 16, 16)]
                word = plsc.load_gather(table_v, [iv])
                vals = plsc.bitcast(word & himask, jnp.float32)
                if t % 2 == 0:
                    acc0 = acc0 + vals
                else:
                    acc1 = acc1 + vals
            acc_v[pl.ds(g * 16, 16)] = acc0 + acc1
            return carry

        lax.fori_loop(0, RPW // 16, title_group, 0)

        def desc_chunk(buf):
            def group(g, c2):
                acc0 = jnp.zeros((16,), jnp.float32)
                acc1 = jnp.zeros((16,), jnp.float32)
                for t in range(DCH):
                    iv = buf[t, pl.ds(g * 16, 16)]
                    word = plsc.load_gather(table_v, [iv])
                    vals = plsc.bitcast(word << 16, jnp.float32)
                    if t % 2 == 0:
                        acc0 = acc0 + vals
                    else:
                        acc1 = acc1 + vals
                acc_v[pl.ds(g * 16, 16)] = (acc_v[pl.ds(g * 16, 16)]
                                            + (acc0 + acc1))
                return c2

            lax.fori_loop(0, RPW // 16, group, 0)

        def pair(p, carry):
            c = 2 * p
            pltpu.make_async_copy(
                descT_hbm.at[pl.ds(0, DCH), pl.ds(r0, RPW)], buf_a,
                semA).wait()
            desc_chunk(buf_a)
            pltpu.make_async_copy(
                descT_hbm.at[pl.ds((c + 2) * DCH, DCH), pl.ds(r0, RPW)],
                buf_a, semA).start()

            pltpu.make_async_copy(
                descT_hbm.at[pl.ds(0, DCH), pl.ds(r0, RPW)], buf_b,
                semB).wait()
            desc_chunk(buf_b)

            @pl.when(p < NCH // 2 - 1)
            def _():
                pltpu.make_async_copy(
                    descT_hbm.at[pl.ds((c + 3) * DCH, DCH),
                                 pl.ds(r0, RPW)], buf_b, semB).start()

            return carry

        lax.fori_loop(0, NCH // 2, pair, 0)

        pltpu.make_async_copy(
            descT_hbm.at[pl.ds(0, DCH), pl.ds(r0, RPW)], buf_a,
            semA).wait()
        desc_chunk(buf_a)

        pltpu.sync_copy(acc_v, out_hbm.at[pl.ds(r0, RPW)])

    return k(s2, titleT, descT)


def _cat_body(catT_ref, wcatT_ref, bcat_ref, wh_ref, bout_ref, out_ref):
    h = lax.dot_general(
        wcatT_ref[...], catT_ref[...],
        dimension_numbers=(((0,), (0,)), ((), ())),
        preferred_element_type=jnp.float32)
    h = jnp.maximum(h + bcat_ref[...], 0.0)
    o = lax.dot_general(
        wh_ref[...], h,
        dimension_numbers=(((1,), (0,)), ((), ())),
        preferred_element_type=jnp.float32)
    out_ref[...] = o + bout_ref[0, 0]


def _cat_part(CatT, WcatT, b_cat2, wh2, b_out2):
    BB = 2048
    return pl.pallas_call(
        _cat_body,
        grid=(B // BB,),
        in_specs=[
            pl.BlockSpec((C, BB), lambda i: (0, i)),
            pl.BlockSpec((C, H), lambda i: (0, 0)),
            pl.BlockSpec((H, 1), lambda i: (0, 0)),
            pl.BlockSpec((1, H), lambda i: (0, 0)),
            pl.BlockSpec((1, 1), lambda i: (0, 0)),
        ],
        out_specs=pl.BlockSpec((1, BB), lambda i: (0, i)),
        out_shape=jax.ShapeDtypeStruct((1, B), jnp.float32),
    )(CatT, WcatT, b_cat2, wh2, b_out2)


def _add_body(a_ref, b_ref, out_ref):
    out_ref[...] = a_ref[...] + b_ref[...]


def _add_final(cat2, emb2):
    BB = 8192
    return pl.pallas_call(
        _add_body,
        grid=(B // BB,),
        in_specs=[
            pl.BlockSpec((1, BB), lambda i: (0, i)),
            pl.BlockSpec((1, BB), lambda i: (0, i)),
        ],
        out_specs=pl.BlockSpec((1, BB), lambda i: (0, i)),
        out_shape=jax.ShapeDtypeStruct((1, B), jnp.float32),
    )(cat2, emb2)


def kernel(Title, FullDescription, Categorical, embedding_matrix, W_cat,
           b_cat, W_out, b_out):
    w2 = jnp.stack([W_out[0, :D] * (1.0 / LT),
                    W_out[0, D:2 * D] * (1.0 / LD)], axis=0)
    s2 = _compute_scores(embedding_matrix.T, w2)

    emb_part = _sc_pool(s2, Title.T.astype(jnp.int32),
                        FullDescription.T.astype(jnp.int32))

    cat2 = _cat_part(Categorical.T, W_cat.T, b_cat.reshape(H, 1),
                     W_out[0:1, 2 * D:], b_out.reshape(1, 1))
    out2 = _add_final(cat2, emb_part.reshape(1, B))
    return out2.reshape(B)


# speedup vs baseline: 1.0716x; 1.0004x over previous
import functools

import jax
import jax.numpy as jnp
from jax import lax
from jax.experimental import pallas as pl
from jax.experimental.pallas import tpu as pltpu
from jax.experimental.pallas import tpu_sc as plsc

B = 16384
LT = 20
LD = 200
V = 100000
D = 64
C = 100
H = 128

NC = 2
NS = 16
NW = NC * NS
RPW = B // NW
DCH = 8
NCH = LD // DCH


def _scores_body(et_ref, w2_ref, out_ref):
    s = lax.dot_general(
        w2_ref[...], et_ref[...],
        dimension_numbers=(((1,), (0,)), ((), ())),
        preferred_element_type=jnp.float32)
    st = lax.bitcast_convert_type(
        s[0:1, :].astype(jnp.bfloat16), jnp.uint16).astype(jnp.uint32)
    sd = lax.bitcast_convert_type(
        s[1:2, :].astype(jnp.bfloat16), jnp.uint16).astype(jnp.uint32)
    packed = (st << 16) | sd
    out_ref[...] = lax.bitcast_convert_type(packed, jnp.int32)


def _compute_scores(ET, w2):
    VB = 12800
    return pl.pallas_call(
        _scores_body,
        grid=(pl.cdiv(V, VB),),
        in_specs=[
            pl.BlockSpec((D, VB), lambda i: (0, i)),
            pl.BlockSpec((2, D), lambda i: (0, 0)),
        ],
        out_specs=pl.BlockSpec((1, VB), lambda i: (0, i)),
        out_shape=jax.ShapeDtypeStruct((1, V), jnp.int32),
    )(ET, w2)


def _sc_pool(s2, titleT, descT):
    mesh = plsc.VectorSubcoreMesh(core_axis_name="c", subcore_axis_name="s")

    @functools.partial(
        pl.kernel,
        mesh=mesh,
        out_type=jax.ShapeDtypeStruct((B,), jnp.float32),
        compiler_params=pltpu.CompilerParams(needs_layout_passes=False),
        scratch_types=[
            pltpu.VMEM((V,), jnp.int32),
            pltpu.VMEM((LT, RPW), jnp.int32),
            pltpu.VMEM((DCH, RPW), jnp.int32),
            pltpu.VMEM((DCH, RPW), jnp.int32),
            pltpu.VMEM((RPW,), jnp.float32),
            pltpu.SemaphoreType.DMA,
            pltpu.SemaphoreType.DMA,
            pltpu.SemaphoreType.DMA,
        ],
    )
    def k(s2_hbm, titleT_hbm, descT_hbm, out_hbm, table_v, tidx_v, buf_a,
          buf_b, acc_v, semT, semA, semB):
        w = lax.axis_index("c") * NS + lax.axis_index("s")
        r0 = w * RPW
        himask = jnp.full((16,), -65536, jnp.int32)

        tcp = pltpu.make_async_copy(s2_hbm.at[0], table_v, semT)
        tcp.start()
        ti = pltpu.make_async_copy(
            titleT_hbm.at[:, pl.ds(r0, RPW)], tidx_v, semT)
        ti.start()
        pltpu.make_async_copy(
            descT_hbm.at[pl.ds(0, DCH), pl.ds(r0, RPW)], buf_a,
            semA).start()
        pltpu.make_async_copy(
            descT_hbm.at[pl.ds(DCH, DCH), pl.ds(r0, RPW)], buf_b,
            semB).start()
        tcp.wait()
        ti.wait()

        def title_group(g, carry):
            acc0 = jnp.zeros((16,), jnp.float32)
            acc1 = jnp.zeros((16,), jnp.float32)
            for t in range(LT):
                iv = tidx_v[t, pl.ds(g * 16, 16)]
                word = plsc.load_gather(table_v, [iv])
                vals = plsc.bitcast(word & himask, jnp.float32)
                if t % 2 == 0:
                    acc0 = acc0 + vals
                else:
                    acc1 = acc1 + vals
            acc_v[pl.ds(g * 16, 16)] = acc0 + acc1
            return carry

        lax.fori_loop(0, RPW // 16, title_group, 0)

        def desc_chunk(buf):
            def group(i, c2):
                for half in range(2):
                    g = 2 * i + half
                    acc0 = jnp.zeros((16,), jnp.float32)
                    acc1 = jnp.zeros((16,), jnp.float32)
                    for t in range(DCH):
                        iv = buf[t, pl.ds(g * 16, 16)]
                        word = plsc.load_gather(table_v, [iv])
                        vals = plsc.bitcast(word << 16, jnp.float32)
                        if t % 2 == 0:
                            acc0 = acc0 + vals
                        else:
                            acc1 = acc1 + vals
                    acc_v[pl.ds(g * 16, 16)] = (acc_v[pl.ds(g * 16, 16)]
                                                + (acc0 + acc1))
                return c2

            lax.fori_loop(0, RPW // 32, group, 0)

        def pair(p, carry):
            c = 2 * p
            pltpu.make_async_copy(
                descT_hbm.at[pl.ds(0, DCH), pl.ds(r0, RPW)], buf_a,
                semA).wait()
            desc_chunk(buf_a)
            pltpu.make_async_copy(
                descT_hbm.at[pl.ds((c + 2) * DCH, DCH), pl.ds(r0, RPW)],
                buf_a, semA).start()

            pltpu.make_async_copy(
                descT_hbm.at[pl.ds(0, DCH), pl.ds(r0, RPW)], buf_b,
                semB).wait()
            desc_chunk(buf_b)

            @pl.when(p < NCH // 2 - 1)
            def _():
                pltpu.make_async_copy(
                    descT_hbm.at[pl.ds((c + 3) * DCH, DCH),
                                 pl.ds(r0, RPW)], buf_b, semB).start()

            return carry

        lax.fori_loop(0, NCH // 2, pair, 0)

        pltpu.make_async_copy(
            descT_hbm.at[pl.ds(0, DCH), pl.ds(r0, RPW)], buf_a,
            semA).wait()
        desc_chunk(buf_a)

        pltpu.sync_copy(acc_v, out_hbm.at[pl.ds(r0, RPW)])

    return k(s2, titleT, descT)


def _cat_body(catT_ref, wcatT_ref, bcat_ref, wh_ref, bout_ref, out_ref):
    h = lax.dot_general(
        wcatT_ref[...], catT_ref[...],
        dimension_numbers=(((0,), (0,)), ((), ())),
        preferred_element_type=jnp.float32)
    h = jnp.maximum(h + bcat_ref[...], 0.0)
    o = lax.dot_general(
        wh_ref[...], h,
        dimension_numbers=(((1,), (0,)), ((), ())),
        preferred_element_type=jnp.float32)
    out_ref[...] = o + bout_ref[0, 0]


def _cat_part(CatT, WcatT, b_cat2, wh2, b_out2):
    BB = 2048
    return pl.pallas_call(
        _cat_body,
        grid=(B // BB,),
        in_specs=[
            pl.BlockSpec((C, BB), lambda i: (0, i)),
            pl.BlockSpec((C, H), lambda i: (0, 0)),
            pl.BlockSpec((H, 1), lambda i: (0, 0)),
            pl.BlockSpec((1, H), lambda i: (0, 0)),
            pl.BlockSpec((1, 1), lambda i: (0, 0)),
        ],
        out_specs=pl.BlockSpec((1, BB), lambda i: (0, i)),
        out_shape=jax.ShapeDtypeStruct((1, B), jnp.float32),
    )(CatT, WcatT, b_cat2, wh2, b_out2)


def _add_body(a_ref, b_ref, out_ref):
    out_ref[...] = a_ref[...] + b_ref[...]


def _add_final(cat2, emb2):
    BB = 8192
    return pl.pallas_call(
        _add_body,
        grid=(B // BB,),
        in_specs=[
            pl.BlockSpec((1, BB), lambda i: (0, i)),
            pl.BlockSpec((1, BB), lambda i: (0, i)),
        ],
        out_specs=pl.BlockSpec((1, BB), lambda i: (0, i)),
        out_shape=jax.ShapeDtypeStruct((1, B), jnp.float32),
    )(cat2, emb2)


def kernel(Title, FullDescription, Categorical, embedding_matrix, W_cat,
           b_cat, W_out, b_out):
    w2 = jnp.stack([W_out[0, :D] * (1.0 / LT),
                    W_out[0, D:2 * D] * (1.0 / LD)], axis=0)
    s2 = _compute_scores(embedding_matrix.T, w2)

    emb_part = _sc_pool(s2, Title.T.astype(jnp.int32),
                        FullDescription.T.astype(jnp.int32))

    cat2 = _cat_part(Categorical.T, W_cat.T, b_cat.reshape(H, 1),
                     W_out[0:1, 2 * D:], b_out.reshape(1, 1))
    out2 = _add_final(cat2, emb_part.reshape(1, B))
    return out2.reshape(B)
